# baseline (speedup 1.0000x reference)
import jax
import jax.numpy as jnp
from jax.experimental import pallas as pl
from jax.experimental.pallas import tpu as pltpu

_N = 768
_D = 256
_F32 = jnp.float32
_BF16 = jnp.bfloat16


def _tdot(a, b):
    return jax.lax.dot_general(
        a, b, (((0,), (0,)), ((), ())), preferred_element_type=_F32)


def _topo_kernel(edges_h, nodes_h, w1_h, wq_h, wk_h, wv_h, wo_h,
                 b1_ref, bq_ref, bk_ref, bv_ref, bo_ref, out_ref,
                 edges_v, nodes_v, w1_v, wq_v, wk_v, wv_v, wo_v,
                 se, sn, s1, sq, sk, sv, so):
    ce = pltpu.make_async_copy(edges_h, edges_v, se)
    ce.start()
    cn = pltpu.make_async_copy(nodes_h, nodes_v, sn)
    cn.start()
    c1 = pltpu.make_async_copy(w1_h, w1_v, s1)
    c1.start()
    cq = pltpu.make_async_copy(wq_h, wq_v, sq)
    cq.start()
    ck = pltpu.make_async_copy(wk_h, wk_v, sk)
    ck.start()
    cv = pltpu.make_async_copy(wv_h, wv_v, sv)
    cv.start()
    co = pltpu.make_async_copy(wo_h, wo_v, so)
    co.start()

    cn.wait()
    c1.wait()
    nodes = nodes_v[...]
    h = jnp.dot(nodes.astype(_BF16), w1_v[...].astype(_BF16),
                preferred_element_type=_F32)

    ce.wait()
    ii = jax.lax.broadcasted_iota(jnp.int32, (_N, _N), 0)
    jj = jax.lax.broadcasted_iota(jnp.int32, (_N, _N), 1)
    eu = jnp.where(ii < jj, edges_v[...], 0.0).astype(_BF16)

    deg = _tdot(eu, jnp.ones((_N, 1), _BF16)) + 1.0
    dinv = jax.lax.rsqrt(deg)
    g = dinv * h
    agg = dinv * (_tdot(eu, g.astype(_BF16)) + g) + b1_ref[...]
    x = nodes + jnp.maximum(agg, 0.0)
    xb = x.astype(_BF16)

    scale = 1.0 / jnp.sqrt(jnp.asarray(_D, _F32))
    cq.wait()
    q = (jnp.dot(xb, wq_v[...].astype(_BF16),
                 preferred_element_type=_F32) + bq_ref[...]) * scale
    ck.wait()
    k = jnp.dot(xb, wk_v[...].astype(_BF16),
                preferred_element_type=_F32) + bk_ref[...]
    cv.wait()
    v = jnp.dot(xb, wv_v[...].astype(_BF16),
                preferred_element_type=_F32) + bv_ref[...]

    logits = jax.lax.dot_general(
        q.astype(_BF16), k.astype(_BF16),
        (((1,), (1,)), ((), ())), preferred_element_type=_F32)
    m = jnp.max(logits, axis=1, keepdims=True)
    p = jnp.exp(logits - m)
    s = jnp.sum(p, axis=1, keepdims=True)
    av = jnp.dot(p.astype(_BF16), v.astype(_BF16),
                 preferred_element_type=_F32) / s
    co.wait()
    out_ref[...] = jnp.dot(av.astype(_BF16), wo_v[...].astype(_BF16),
                           preferred_element_type=_F32) + bo_ref[...]


def kernel(nodes, edges, W1, b1, Wq, bq, Wk, bk, Wv, bv, Wo, bo):
    b1r, bqr, bkr, bvr, bor = (b.reshape(1, _D) for b in (b1, bq, bk, bv, bo))
    any_spec = pl.BlockSpec(memory_space=pltpu.MemorySpace.HBM)
    vmem_spec = pl.BlockSpec(memory_space=pltpu.MemorySpace.VMEM)
    return pl.pallas_call(
        _topo_kernel,
        out_shape=jax.ShapeDtypeStruct((_N, _D), jnp.float32),
        in_specs=[any_spec] * 7 + [vmem_spec] * 5,
        out_specs=vmem_spec,
        scratch_shapes=[
            pltpu.VMEM((_N, _N), _F32),
            pltpu.VMEM((_N, _D), _F32),
            pltpu.VMEM((_D, _D), _F32),
            pltpu.VMEM((_D, _D), _F32),
            pltpu.VMEM((_D, _D), _F32),
            pltpu.VMEM((_D, _D), _F32),
            pltpu.VMEM((_D, _D), _F32),
        ] + [pltpu.SemaphoreType.DMA] * 7,
    )(edges, nodes, W1, Wq, Wk, Wv, Wo, b1r, bqr, bkr, bvr, bor)


# speedup vs baseline: 1.2286x; 1.2286x over previous
import jax
import jax.numpy as jnp
from jax.experimental import pallas as pl

_N = 768
_D = 256
_F32 = jnp.float32
_BF16 = jnp.bfloat16


def _tdot(a, b):
    return jax.lax.dot_general(
        a, b, (((0,), (0,)), ((), ())), preferred_element_type=_F32)


def _topo_kernel(edges_ref, nodes_ref, w1_ref, b1_ref, wq_ref, bq_ref,
                 wk_ref, bk_ref, wv_ref, bv_ref, wo_ref, bo_ref, out_ref):
    ii = jax.lax.broadcasted_iota(jnp.int32, (_N, _N), 0)
    jj = jax.lax.broadcasted_iota(jnp.int32, (_N, _N), 1)
    eu = jnp.where(ii < jj, edges_ref[...], 0.0).astype(_BF16)

    deg = _tdot(eu, jnp.ones((_N, 1), _BF16)) + 1.0
    dinv = jax.lax.rsqrt(deg)

    nodes = nodes_ref[...]
    h = jnp.dot(nodes.astype(_BF16), w1_ref[...].astype(_BF16),
                preferred_element_type=_F32)
    g = dinv * h
    agg = dinv * (_tdot(eu, g.astype(_BF16)) + g) + b1_ref[...]
    x = nodes + jnp.maximum(agg, 0.0)
    xb = x.astype(_BF16)

    scale = 1.0 / jnp.sqrt(jnp.asarray(_D, _F32))
    q = (jnp.dot(xb, wq_ref[...].astype(_BF16),
                 preferred_element_type=_F32) + bq_ref[...]) * scale
    k = jnp.dot(xb, wk_ref[...].astype(_BF16),
                preferred_element_type=_F32) + bk_ref[...]
    v = jnp.dot(xb, wv_ref[...].astype(_BF16),
                preferred_element_type=_F32) + bv_ref[...]

    logits = jax.lax.dot_general(
        q.astype(_BF16), k.astype(_BF16),
        (((1,), (1,)), ((), ())), preferred_element_type=_F32)
    m = jnp.max(logits, axis=1, keepdims=True)
    p = jnp.exp(logits - m)
    s = jnp.sum(p, axis=1, keepdims=True)
    av = jnp.dot(p.astype(_BF16), v.astype(_BF16),
                 preferred_element_type=_F32) / s
    out_ref[...] = jnp.dot(av.astype(_BF16), wo_ref[...].astype(_BF16),
                           preferred_element_type=_F32) + bo_ref[...]


def kernel(nodes, edges, W1, b1, Wq, bq, Wk, bk, Wv, bv, Wo, bo):
    b1r, bqr, bkr, bvr, bor = (b.reshape(1, _D) for b in (b1, bq, bk, bv, bo))
    return pl.pallas_call(
        _topo_kernel,
        out_shape=jax.ShapeDtypeStruct((_N, _D), jnp.float32),
    )(edges, nodes, W1, b1r, Wq, bqr, Wk, bkr, Wv, bvr, Wo, bor)


# speedup vs baseline: 1.2438x; 1.0124x over previous
import jax
import jax.numpy as jnp
from jax.experimental import pallas as pl

_N = 768
_D = 256
_F32 = jnp.float32
_BF16 = jnp.bfloat16


def _tdot(a, b):
    return jax.lax.dot_general(
        a, b, (((0,), (0,)), ((), ())), preferred_element_type=_F32)


def _topo_kernel(edges_ref, nodes_ref, w1_ref, b1_ref, wq_ref, bq_ref,
                 wk_ref, bk_ref, wv_ref, bv_ref, wo_ref, bo_ref, out_ref):
    ii = jax.lax.broadcasted_iota(jnp.int32, (_N, _N), 0)
    jj = jax.lax.broadcasted_iota(jnp.int32, (_N, _N), 1)
    eu = jnp.where(ii < jj, edges_ref[...], 0.0).astype(_BF16)

    deg = _tdot(eu, jnp.ones((_N, 1), _BF16)) + 1.0
    dinv = jax.lax.rsqrt(deg)

    nodes = nodes_ref[...]
    h = jnp.dot(nodes.astype(_BF16), w1_ref[...].astype(_BF16),
                preferred_element_type=_F32)
    g = dinv * h
    agg = dinv * (_tdot(eu, g.astype(_BF16)) + g) + b1_ref[...]
    x = nodes + jnp.maximum(agg, 0.0)
    xb = x.astype(_BF16)

    scale = 1.0 / jnp.sqrt(jnp.asarray(_D, _F32))
    q = (jnp.dot(xb, wq_ref[...].astype(_BF16),
                 preferred_element_type=_F32) + bq_ref[...]) * scale
    k = jnp.dot(xb, wk_ref[...].astype(_BF16),
                preferred_element_type=_F32) + bk_ref[...]
    v = jnp.dot(xb, wv_ref[...].astype(_BF16),
                preferred_element_type=_F32) + bv_ref[...]

    logits = jax.lax.dot_general(
        q.astype(_BF16), k.astype(_BF16),
        (((1,), (1,)), ((), ())), preferred_element_type=_F32)
    m = jnp.max(logits, axis=1, keepdims=True)
    p = jnp.exp(logits - m).astype(_BF16)
    s = jnp.dot(p, jnp.ones((_N, 1), _BF16), preferred_element_type=_F32)
    av = jnp.dot(p, v.astype(_BF16), preferred_element_type=_F32) / s
    out_ref[...] = jnp.dot(av.astype(_BF16), wo_ref[...].astype(_BF16),
                           preferred_element_type=_F32) + bo_ref[...]


def kernel(nodes, edges, W1, b1, Wq, bq, Wk, bk, Wv, bv, Wo, bo):
    b1r, bqr, bkr, bvr, bor = (b.reshape(1, _D) for b in (b1, bq, bk, bv, bo))
    return pl.pallas_call(
        _topo_kernel,
        out_shape=jax.ShapeDtypeStruct((_N, _D), jnp.float32),
    )(edges, nodes, W1, b1r, Wq, bqr, Wk, bkr, Wv, bvr, Wo, bor)


# speedup vs baseline: 1.2978x; 1.0434x over previous
import jax
import jax.numpy as jnp
from jax.experimental import pallas as pl

_N = 768
_D = 256
_F32 = jnp.float32
_BF16 = jnp.bfloat16


def _tdot(a, b):
    return jax.lax.dot_general(
        a, b, (((0,), (0,)), ((), ())), preferred_element_type=_F32)


def _topo_kernel(edges_ref, nodes_ref, w1_ref, b1_ref, wq_ref, bq_ref,
                 wk_ref, bk_ref, wv_ref, bv_ref, wo_ref, bo_ref, out_ref):
    nodes = nodes_ref[...]
    h = jnp.dot(nodes.astype(_BF16), w1_ref[...].astype(_BF16),
                preferred_element_type=_F32)

    ii = jax.lax.broadcasted_iota(jnp.int32, (_N, _N), 0)
    jj = jax.lax.broadcasted_iota(jnp.int32, (_N, _N), 1)
    eu = jnp.where(ii < jj, edges_ref[...], 0.0).astype(_BF16)

    deg = _tdot(eu, jnp.ones((_N, 1), _BF16)) + 1.0
    dinv = jax.lax.rsqrt(deg)
    g = dinv * h
    agg = dinv * (_tdot(eu, g.astype(_BF16)) + g) + b1_ref[...]
    x = nodes + jnp.maximum(agg, 0.0)
    xb = x.astype(_BF16)

    scale = 1.0 / jnp.sqrt(jnp.asarray(_D, _F32))
    q = (jnp.dot(xb, wq_ref[...].astype(_BF16),
                 preferred_element_type=_F32) + bq_ref[...]) * scale
    k = jnp.dot(xb, wk_ref[...].astype(_BF16),
                preferred_element_type=_F32) + bk_ref[...]
    v = jnp.dot(xb, wv_ref[...].astype(_BF16),
                preferred_element_type=_F32) + bv_ref[...]

    logits = jax.lax.dot_general(
        q.astype(_BF16), k.astype(_BF16),
        (((1,), (1,)), ((), ())), preferred_element_type=_F32)
    m = jnp.max(logits, axis=1, keepdims=True)
    p = jnp.exp(logits - m).astype(_BF16)
    s = jnp.dot(p, jnp.ones((_N, 1), _BF16), preferred_element_type=_F32)
    av = jnp.dot(p, v.astype(_BF16), preferred_element_type=_F32) / s
    out_ref[...] = jnp.dot(av.astype(_BF16), wo_ref[...].astype(_BF16),
                           preferred_element_type=_F32) + bo_ref[...]


def kernel(nodes, edges, W1, b1, Wq, bq, Wk, bk, Wv, bv, Wo, bo):
    b1r, bqr, bkr, bvr, bor = (b.reshape(1, _D) for b in (b1, bq, bk, bv, bo))
    return pl.pallas_call(
        _topo_kernel,
        out_shape=jax.ShapeDtypeStruct((_N, _D), jnp.float32),
    )(edges, nodes, W1, b1r, Wq, bqr, Wk, bkr, Wv, bvr, Wo, bor)


# speedup vs baseline: 1.3479x; 1.0386x over previous
import jax
import jax.numpy as jnp
from jax.experimental import pallas as pl

_N = 768
_D = 256
_F32 = jnp.float32
_BF16 = jnp.bfloat16


def _tdot(a, b):
    return jax.lax.dot_general(
        a, b, (((0,), (0,)), ((), ())), preferred_element_type=_F32)


def _topo_kernel(edges_ref, nodes_ref, w1_ref, b1_ref, wq_ref, bq_ref,
                 wk_ref, bk_ref, wv_ref, bv_ref, wo_ref, bo_ref, out_ref):
    nodes = nodes_ref[...]
    h = jnp.dot(nodes.astype(_BF16), w1_ref[...].astype(_BF16),
                preferred_element_type=_F32)

    ii = jax.lax.broadcasted_iota(jnp.int32, (_N, _N), 0)
    jj = jax.lax.broadcasted_iota(jnp.int32, (_N, _N), 1)
    eu_f = jnp.where(ii < jj, edges_ref[...], 0.0)
    eu = eu_f.astype(_BF16)

    deg_row = jnp.sum(eu_f, axis=0, keepdims=True) + 1.0
    dinv = jax.lax.rsqrt(deg_row).reshape(_N, 1)
    g = dinv * h
    agg = dinv * (_tdot(eu, g.astype(_BF16)) + g) + b1_ref[...]
    x = nodes + jnp.maximum(agg, 0.0)
    xb = x.astype(_BF16)

    scale = 1.0 / jnp.sqrt(jnp.asarray(_D, _F32))
    q = (jnp.dot(xb, wq_ref[...].astype(_BF16),
                 preferred_element_type=_F32) + bq_ref[...]) * scale
    k = jnp.dot(xb, wk_ref[...].astype(_BF16),
                preferred_element_type=_F32) + bk_ref[...]
    v = jnp.dot(xb, wv_ref[...].astype(_BF16),
                preferred_element_type=_F32) + bv_ref[...]

    logits = jax.lax.dot_general(
        q.astype(_BF16), k.astype(_BF16),
        (((1,), (1,)), ((), ())), preferred_element_type=_F32)
    m = jnp.max(logits, axis=1, keepdims=True)
    p = jnp.exp(logits - m).astype(_BF16)
    s = jnp.dot(p, jnp.ones((_N, 1), _BF16), preferred_element_type=_F32)
    av = jnp.dot(p, v.astype(_BF16), preferred_element_type=_F32) / s
    out_ref[...] = jnp.dot(av.astype(_BF16), wo_ref[...].astype(_BF16),
                           preferred_element_type=_F32) + bo_ref[...]


def kernel(nodes, edges, W1, b1, Wq, bq, Wk, bk, Wv, bv, Wo, bo):
    b1r, bqr, bkr, bvr, bor = (b.reshape(1, _D) for b in (b1, bq, bk, bv, bo))
    return pl.pallas_call(
        _topo_kernel,
        out_shape=jax.ShapeDtypeStruct((_N, _D), jnp.float32),
    )(edges, nodes, W1, b1r, Wq, bqr, Wk, bkr, Wv, bvr, Wo, bor)


# speedup vs baseline: 1.3883x; 1.0300x over previous
import jax
import jax.numpy as jnp
from jax.experimental import pallas as pl

_N = 768
_D = 256
_F32 = jnp.float32
_BF16 = jnp.bfloat16


def _tdot(a, b):
    return jax.lax.dot_general(
        a, b, (((0,), (0,)), ((), ())), preferred_element_type=_F32)


def _topo_kernel(edges_ref, nodes_ref, w1_ref, b1_ref, wq_ref, bq_ref,
                 wk_ref, bk_ref, wv_ref, bv_ref, wo_ref, bo_ref, out_ref):
    wqb = wq_ref[...].astype(_BF16)
    wkb = wk_ref[...].astype(_BF16)
    wvb = wv_ref[...].astype(_BF16)
    wob = wo_ref[...].astype(_BF16)
    nodes = nodes_ref[...]
    h = jnp.dot(nodes.astype(_BF16), w1_ref[...].astype(_BF16),
                preferred_element_type=_F32)

    ii = jax.lax.broadcasted_iota(jnp.int32, (_N, _N), 0)
    jj = jax.lax.broadcasted_iota(jnp.int32, (_N, _N), 1)
    eu_f = jnp.where(ii < jj, edges_ref[...], 0.0)
    eu = eu_f.astype(_BF16)

    deg_row = jnp.sum(eu_f, axis=0, keepdims=True) + 1.0
    dinv = jax.lax.rsqrt(deg_row).reshape(_N, 1)
    g = dinv * h
    agg_t = jax.lax.dot_general(
        g.astype(_BF16), eu, (((0,), (0,)), ((), ())),
        preferred_element_type=_F32)
    agg = dinv * (agg_t.T + g) + b1_ref[...]
    x = nodes + jnp.maximum(agg, 0.0)
    xb = x.astype(_BF16)

    scale = 1.0 / jnp.sqrt(jnp.asarray(_D, _F32))
    q = (jnp.dot(xb, wqb,
                 preferred_element_type=_F32) + bq_ref[...]) * scale
    k = jnp.dot(xb, wkb,
                preferred_element_type=_F32) + bk_ref[...]
    v = jnp.dot(xb, wvb,
                preferred_element_type=_F32) + bv_ref[...]

    logits = jax.lax.dot_general(
        q.astype(_BF16), k.astype(_BF16),
        (((1,), (1,)), ((), ())), preferred_element_type=_F32)
    m = jnp.max(logits, axis=1, keepdims=True)
    p = jnp.exp(logits - m).astype(_BF16)
    s = jnp.dot(p, jnp.ones((_N, 1), _BF16), preferred_element_type=_F32)
    av = jnp.dot(p, v.astype(_BF16), preferred_element_type=_F32) / s
    out_ref[...] = jnp.dot(av.astype(_BF16), wob,
                           preferred_element_type=_F32) + bo_ref[...]


def kernel(nodes, edges, W1, b1, Wq, bq, Wk, bk, Wv, bv, Wo, bo):
    b1r, bqr, bkr, bvr, bor = (b.reshape(1, _D) for b in (b1, bq, bk, bv, bo))
    return pl.pallas_call(
        _topo_kernel,
        out_shape=jax.ShapeDtypeStruct((_N, _D), jnp.float32),
    )(edges, nodes, W1, b1r, Wq, bqr, Wk, bkr, Wv, bvr, Wo, bor)
